# baseline (speedup 1.0000x reference)
import functools

import jax
import jax.numpy as jnp
from jax import lax
from jax.experimental import pallas as pl
from jax.experimental.pallas import tpu as pltpu
from jax.experimental.pallas import tpu_sc as plsc

_EMBED = 32
_CHUNK = 128


@functools.lru_cache(maxsize=None)
def _make_gather(n_rows_total):
    info = plsc.get_sparse_core_info()
    nc, ns = info.num_cores, info.num_subcores
    nw = nc * ns
    assert n_rows_total % (nw * 8) == 0
    b_per_w = n_rows_total // nw
    n_chunks = b_per_w // _CHUNK
    assert n_chunks * _CHUNK == b_per_w

    mesh = plsc.VectorSubcoreMesh(core_axis_name="c", subcore_axis_name="s")

    @functools.partial(
        pl.kernel,
        mesh=mesh,
        compiler_params=pltpu.CompilerParams(use_tc_tiling_on_sc=False),
        out_type=jax.ShapeDtypeStruct((n_rows_total, _EMBED), jnp.float32),
        scratch_types=[
            pltpu.VMEM((b_per_w,), jnp.int32),
            pltpu.VMEM((b_per_w, _EMBED), jnp.float32),
            pltpu.SemaphoreType.DMA,
        ],
    )
    def gather_k(tbl_hbm, idx_hbm, out_hbm, idx_v, rows_v, sem):
        wid = lax.axis_index("s") * nc + lax.axis_index("c")
        base = wid * b_per_w
        pltpu.sync_copy(idx_hbm.at[pl.ds(base, b_per_w)], idx_v)

        def body(j, carry):
            sl = pl.ds(j * _CHUNK, _CHUNK)
            pltpu.async_copy(tbl_hbm.at[idx_v.at[sl]], rows_v.at[sl, :], sem).wait()
            return carry

        lax.fori_loop(0, n_chunks, body, 0)
        pltpu.sync_copy(rows_v, out_hbm.at[pl.ds(base, b_per_w)])

    return gather_k


def _mlp_body(*refs):
    r_ref = refs[0]
    out_ref = refs[-1]
    w = refs[1:-1]
    r = r_ref[...]
    n_units = (len(w) - 2) // 4
    for u in range(n_units):
        w1, b1, w2, b2 = w[4 * u : 4 * u + 4]
        h = jnp.dot(r, w1[...], preferred_element_type=jnp.float32) + b1[...]
        h = jnp.maximum(h, 0.0)
        h = jnp.dot(h, w2[...], preferred_element_type=jnp.float32) + b2[...]
        r = jnp.maximum(r + h, 0.0)
    wd, bd = w[-2], w[-1]
    logit = jnp.dot(r, wd[...], preferred_element_type=jnp.float32) + bd[...]
    out_ref[...] = jax.nn.sigmoid(logit)


def _mlp(r, flat_w, block_b=512):
    batch, stack = r.shape
    grid = (batch // block_b,)
    full = lambda a: pl.BlockSpec(a.shape, lambda i: (0,) * a.ndim)
    in_specs = [pl.BlockSpec((block_b, stack), lambda i: (i, 0))]
    in_specs += [full(a) for a in flat_w]
    return pl.pallas_call(
        _mlp_body,
        grid=grid,
        in_specs=in_specs,
        out_specs=pl.BlockSpec((block_b, 1), lambda i: (i, 0)),
        out_shape=jax.ShapeDtypeStruct((batch, 1), jnp.float32),
    )(r, *flat_w)


def kernel(sparse_inputs, params):
    tables = params["tables"]
    n_fields, vocab, embed = tables.shape
    batch = sparse_inputs.shape[0]
    flat_tbl = tables.reshape(n_fields * vocab, embed)
    offs = (jnp.arange(n_fields, dtype=jnp.int32) * vocab)[None, :]
    flat_idx = (sparse_inputs.astype(jnp.int32) + offs).reshape(-1)

    rows = _make_gather(batch * n_fields)(flat_tbl, flat_idx)
    r = rows.reshape(batch, n_fields * embed)

    flat_w = []
    for (w1, b1, w2, b2) in params["res"]:
        flat_w += [w1, b1[None, :], w2, b2[None, :]]
    flat_w += [params["Wd"], params["bd"][None, :]]
    return _mlp(r, tuple(flat_w))
